# Initial kernel scaffold; baseline (speedup 1.0000x reference)
import jax
import jax.numpy as jnp
from jax.experimental import pallas as pl


def kernel(idx, x, table):
    raise NotImplementedError("write your pallas kernel here")


# speedup vs baseline: 2.6041x; 2.6041x over previous
import functools

import jax
import jax.numpy as jnp
from jax import lax
from jax.experimental import pallas as pl
from jax.experimental.pallas import tpu as pltpu
from jax.experimental.pallas import tpu_sc as plsc

EMB_DIM = 128
CHUNK = 128


@functools.lru_cache(maxsize=None)
def _make_lookup(n_idx: int, n_emb: int, d: int):
    info = plsc.get_sparse_core_info()
    nw = info.num_cores * info.num_subcores
    assert n_idx % (nw * CHUNK) == 0
    per_w = n_idx // nw
    n_chunks = per_w // CHUNK
    mesh = plsc.VectorSubcoreMesh(core_axis_name="c", subcore_axis_name="s")

    @functools.partial(
        pl.kernel,
        mesh=mesh,
        out_type=jax.ShapeDtypeStruct((n_idx, d), jnp.float32),
        scratch_types=[
            pltpu.VMEM((1, CHUNK), jnp.int32),
            pltpu.VMEM((1, CHUNK, d), jnp.float32),
            pltpu.SemaphoreType.DMA,
        ],
    )
    def lookup(table_hbm, idx_hbm, out_hbm, idx_v, rows_v, gsem):
        wid = lax.axis_index("s") * info.num_cores + lax.axis_index("c")
        base = wid * per_w

        def body(i, carry):
            start = base + i * CHUNK
            pltpu.sync_copy(idx_hbm.at[pl.ds(start, CHUNK)], idx_v.at[0])
            pltpu.async_copy(table_hbm.at[idx_v.at[0]], rows_v.at[0], gsem).wait()
            pltpu.sync_copy(rows_v.at[0], out_hbm.at[pl.ds(start, CHUNK)])
            return carry

        lax.fori_loop(0, n_chunks, body, 0)

    return lookup


def kernel(idx, x, table):
    del x
    b, l = idx.shape
    n = b * l
    idx_flat = idx.reshape(n).astype(jnp.int32)
    lookup = _make_lookup(n, table.shape[0], table.shape[1])
    out = lookup(table.astype(jnp.float32), idx_flat)
    return out.reshape(b, l, table.shape[1])


# speedup vs baseline: 2.6523x; 1.0185x over previous
import functools

import jax
import jax.numpy as jnp
from jax import lax
from jax.experimental import pallas as pl
from jax.experimental.pallas import tpu as pltpu
from jax.experimental.pallas import tpu_sc as plsc

EMB_DIM = 128
CHUNK = 128


@functools.lru_cache(maxsize=None)
def _make_lookup(n_idx: int, n_emb: int, d: int):
    info = plsc.get_sparse_core_info()
    nw = info.num_cores * info.num_subcores
    assert n_idx % (nw * 2 * CHUNK) == 0
    per_w = n_idx // nw
    n_chunks = per_w // CHUNK
    mesh = plsc.VectorSubcoreMesh(core_axis_name="c", subcore_axis_name="s")

    @functools.partial(
        pl.kernel,
        mesh=mesh,
        out_type=jax.ShapeDtypeStruct((n_idx, d), jnp.float32),
        scratch_types=[
            pltpu.VMEM((2, CHUNK), jnp.int32),
            pltpu.VMEM((2, CHUNK, d), jnp.float32),
            pltpu.SemaphoreType.DMA,
            pltpu.SemaphoreType.DMA,
            pltpu.SemaphoreType.DMA,
            pltpu.SemaphoreType.DMA,
        ],
    )
    def lookup(table_hbm, idx_hbm, out_hbm, idx_v, rows_v, g0, g1, o0, o1):
        wid = lax.axis_index("s") * info.num_cores + lax.axis_index("c")
        base = wid * per_w
        gsem = (g0, g1)
        osem = (o0, o1)

        def start_gather(i, b):
            pltpu.sync_copy(idx_hbm.at[pl.ds(base + i * CHUNK, CHUNK)],
                            idx_v.at[b])
            pltpu.async_copy(table_hbm.at[idx_v.at[b]], rows_v.at[b], gsem[b])

        def wait_gather(b):
            pltpu.make_async_copy(table_hbm.at[idx_v.at[b]], rows_v.at[b],
                                  gsem[b]).wait()

        def start_store(i, b):
            pltpu.async_copy(rows_v.at[b],
                             out_hbm.at[pl.ds(base + i * CHUNK, CHUNK)],
                             osem[b])

        def wait_store(i, b):
            pltpu.make_async_copy(rows_v.at[b],
                                  out_hbm.at[pl.ds(base + i * CHUNK, CHUNK)],
                                  osem[b]).wait()

        start_gather(0, 0)
        start_gather(1, 1)
        wait_gather(0)
        start_store(0, 0)

        def body(j, carry):
            i0 = 2 * j
            i1 = i0 + 1
            wait_store(i0 - 2, 0)
            start_gather(i0, 0)
            wait_gather(1)
            start_store(i0 - 1, 1)
            wait_store(i1 - 2, 1)
            start_gather(i1, 1)
            wait_gather(0)
            start_store(i0, 0)
            return carry

        lax.fori_loop(1, n_chunks // 2, body, 0)

        wait_gather(1)
        start_store(n_chunks - 1, 1)
        wait_store(n_chunks - 2, 0)
        wait_store(n_chunks - 1, 1)

    return lookup


def kernel(idx, x, table):
    del x
    b, l = idx.shape
    n = b * l
    idx_flat = idx.reshape(n).astype(jnp.int32)
    lookup = _make_lookup(n, table.shape[0], table.shape[1])
    out = lookup(table.astype(jnp.float32), idx_flat)
    return out.reshape(b, l, table.shape[1])
